# baseline (speedup 1.0000x reference)
import functools

import jax
import jax.numpy as jnp
from jax import lax
from jax.experimental import pallas as pl
from jax.experimental.pallas import tpu as pltpu
from jax.experimental.pallas import tpu_sc as plsc

NC, NS, LANES = 2, 16, 16
NW = NC * NS
CHUNK = 128
ROWBLK = 1024


def _sc_mesh():
    return plsc.VectorSubcoreMesh(core_axis_name="c", subcore_axis_name="s")


def _make_deg_kernel(epw, npad, e, nhalf):
    win = epw // nhalf
    nreal_win = e // win

    @functools.partial(
        pl.kernel,
        out_type=jax.ShapeDtypeStruct((NW, npad), jnp.float32),
        mesh=_sc_mesh(),
        scratch_types=[
            pltpu.VMEM((epw,), jnp.int32),
            pltpu.VMEM((npad,), jnp.float32),
        ],
        compiler_params=pltpu.CompilerParams(needs_layout_passes=False),
    )
    def deg_kernel(ei_hbm, pad_hbm, out_hbm, dst_v, hist_v):
        c = lax.axis_index("c")
        s = lax.axis_index("s")
        w = c * NS + s
        for h in range(nhalf):
            wi = w * nhalf + h
            seg = dst_v.at[pl.ds(h * win, win)]

            @pl.when(wi < nreal_win)
            def _():
                pltpu.sync_copy(ei_hbm.at[1, pl.ds(wi * win, win)], seg)

            @pl.when(wi >= nreal_win)
            def _():
                pltpu.sync_copy(
                    pad_hbm.at[pl.ds((wi - nreal_win) * win, win)], seg)

        def zero_body(i, carry):
            hist_v[pl.ds(i * LANES, LANES)] = jnp.zeros((LANES,), jnp.float32)
            return carry

        lax.fori_loop(0, npad // LANES, zero_body, 0)

        ones = jnp.ones((LANES,), jnp.float32)

        def hist_body(i, carry):
            idx = dst_v[pl.ds(i * LANES, LANES)]
            plsc.addupdate_scatter(hist_v, [idx], ones)
            return carry

        lax.fori_loop(0, epw // LANES, hist_body, 0)
        pltpu.sync_copy(hist_v, out_hbm.at[w])

    return deg_kernel


def _make_agg_kernel(epw, npad, dim, chunk, nbuf, nhalf, e):
    stripe = npad // NS
    nch2 = epw // chunk // nhalf
    win = epw // nhalf
    nreal_win = e // win

    @functools.partial(
        pl.kernel,
        out_type=jax.ShapeDtypeStruct((NC, npad, dim), jnp.float32),
        mesh=_sc_mesh(),
        scratch_types=[
            pltpu.VMEM((nch2 * chunk,), jnp.int32),
            pltpu.VMEM((nch2, chunk), jnp.int32),
            [pltpu.VMEM((chunk, dim), jnp.float32) for _ in range(nbuf)],
            pltpu.VMEM_SHARED((npad, dim), jnp.float32),
            [pltpu.SemaphoreType.DMA for _ in range(nbuf)],
        ],
        compiler_params=pltpu.CompilerParams(needs_layout_passes=False),
    )
    def agg_kernel(g_hbm, src_hbm, dst3_hbm, padf_hbm, pad3_hbm, out_hbm,
                   sidx, didx, bufs, acc, sems):
        c = lax.axis_index("c")
        s = lax.axis_index("s")
        w = c * NS + s
        rows0 = bufs[0]

        def zrow(t, carry):
            rows0[t // (dim // LANES), pl.ds((t % (dim // LANES)) * LANES, LANES)] = (
                jnp.zeros((LANES,), jnp.float32)
            )
            return carry

        lax.fori_loop(0, chunk * (dim // LANES), zrow, 0)

        def zstripe(j, carry):
            pltpu.sync_copy(rows0, acc.at[pl.ds(s * stripe + j * chunk, chunk)])
            return carry

        lax.fori_loop(0, stripe // chunk, zstripe, 0)
        plsc.subcore_barrier()

        for h in range(nhalf):
            wi = w * nhalf + h

            @pl.when(wi < nreal_win)
            def _():
                pltpu.sync_copy(src_hbm.at[pl.ds(wi * win, win)], sidx)
                pltpu.sync_copy(dst3_hbm.at[pl.ds(wi * nch2, nch2)], didx)

            @pl.when(wi >= nreal_win)
            def _():
                pltpu.sync_copy(
                    padf_hbm.at[pl.ds((wi - nreal_win) * win, win)], sidx)
                pltpu.sync_copy(pad3_hbm.at[wi - nreal_win], didx)

            def gidx(i):
                return sidx.at[pl.ds(i * chunk, chunk)]

            for b in range(nbuf):
                pltpu.async_copy(g_hbm.at[gidx(b)], bufs[b], sems[b])

            def chunk_body(j, carry):
                for b in range(nbuf):
                    i = j * nbuf + b
                    pltpu.make_async_copy(
                        g_hbm.at[gidx(i)], bufs[b], sems[b]).wait()
                    pltpu.sync_copy(bufs[b], acc.at[didx.at[i]], add=True)

                    @pl.when(i + nbuf < nch2)
                    def _():
                        pltpu.async_copy(
                            g_hbm.at[gidx(i + nbuf)], bufs[b], sems[b])
                return carry

            lax.fori_loop(0, nch2 // nbuf, chunk_body, 0)
        plsc.subcore_barrier()
        pltpu.sync_copy(
            acc.at[pl.ds(s * stripe, stripe)],
            out_hbm.at[c, pl.ds(s * stripe, stripe)],
        )

    return agg_kernel


def _dinv_from_hist(hist_blk):
    deg = jnp.sum(hist_blk, axis=0) + 1.0
    return lax.rsqrt(deg)


def _make_tc_first(n):
    def _tc_first(hist_ref, x_ref, w_ref, g_ref):
        dinv = _dinv_from_hist(hist_ref[...])
        h = jnp.dot(x_ref[...], w_ref[...], preferred_element_type=jnp.float32)
        rid = (pl.program_id(0) * ROWBLK
               + jax.lax.broadcasted_iota(jnp.int32, (ROWBLK, 1), 0))
        g_ref[...] = jnp.where(rid < n, h * dinv[:, None], 0.0)

    return _tc_first


def _tc_mid(hist_ref, s_ref, g_ref, b_ref, w_ref, g2_ref):
    dinv = _dinv_from_hist(hist_ref[...])
    agg = s_ref[0] + s_ref[1] + g_ref[...]
    act = jnp.maximum(agg * dinv[:, None] + b_ref[...], 0.0)
    h = jnp.dot(act, w_ref[...], preferred_element_type=jnp.float32)
    g2_ref[...] = h * dinv[:, None]


def _tc_last(hist_ref, s_ref, g_ref, b_ref, out_ref):
    dinv = _dinv_from_hist(hist_ref[...])
    agg = s_ref[0] + s_ref[1] + g_ref[...]
    out_ref[...] = agg * dinv[:, None] + b_ref[...]


def kernel(x, edge_index, W1, b1, W2, b2):
    n, dim = x.shape
    e = edge_index.shape[1]
    npad = ((n + ROWBLK) // ROWBLK) * ROWBLK
    chunk, nbuf, nhalf = 64, 5, 4
    chw = nhalf * nbuf * chunk
    epw = ((e + NW * chw - 1) // (NW * chw)) * chw
    epad = NW * epw
    grid = npad // ROWBLK

    win = epw // nhalf
    assert e % win == 0 and e % chunk == 0 and (e // NW) % 8 == 0
    pad_tail = n + jnp.arange(epad - e, dtype=jnp.int32) % (npad - n)
    pad3 = pad_tail.reshape((epad - e) // win, win // chunk, chunk)
    src_flat = edge_index[0]
    dst3 = edge_index[1].reshape(e // chunk, chunk)
    b1r = b1.reshape(1, dim)
    b2r = b2.reshape(1, dim)

    hist = _make_deg_kernel(epw, npad, e, nhalf)(edge_index, pad_tail)
    agg = _make_agg_kernel(epw, npad, dim, chunk, nbuf, nhalf, e)

    hist_spec = pl.BlockSpec((NW, ROWBLK), lambda i: (0, i))
    row_spec = pl.BlockSpec((ROWBLK, dim), lambda i: (i, 0))
    s_spec = pl.BlockSpec((NC, ROWBLK, dim), lambda i: (0, i, 0))
    full_spec = pl.BlockSpec((dim, dim), lambda i: (0, 0))
    bias_spec = pl.BlockSpec((1, dim), lambda i: (0, 0))
    out_type = jax.ShapeDtypeStruct((npad, dim), jnp.float32)

    g1 = pl.pallas_call(
        _make_tc_first(n),
        grid=(grid,),
        in_specs=[hist_spec, row_spec, full_spec],
        out_specs=row_spec,
        out_shape=out_type,
    )(hist, x, W1)

    s1 = agg(g1, src_flat, dst3, pad_tail, pad3)

    g2 = pl.pallas_call(
        _tc_mid,
        grid=(grid,),
        in_specs=[hist_spec, s_spec, row_spec, bias_spec, full_spec],
        out_specs=row_spec,
        out_shape=out_type,
    )(hist, s1, g1, b1r, W2)

    s2 = agg(g2, src_flat, dst3, pad_tail, pad3)

    out = pl.pallas_call(
        _tc_last,
        grid=(grid,),
        in_specs=[hist_spec, s_spec, row_spec, bias_spec],
        out_specs=row_spec,
        out_shape=jax.ShapeDtypeStruct((n, dim), jnp.float32),
    )(hist, s2, g2, b2r)

    return out


# speedup vs baseline: 1.0079x; 1.0079x over previous
import functools

import jax
import jax.numpy as jnp
from jax import lax
from jax.experimental import pallas as pl
from jax.experimental.pallas import tpu as pltpu
from jax.experimental.pallas import tpu_sc as plsc

NC, NS, LANES = 2, 16, 16
NW = NC * NS
CHUNK = 128
ROWBLK = 1024


def _sc_mesh():
    return plsc.VectorSubcoreMesh(core_axis_name="c", subcore_axis_name="s")


def _make_deg_kernel(epw, npad, e, nhalf):
    win = epw // nhalf
    nreal_win = e // win

    @functools.partial(
        pl.kernel,
        out_type=jax.ShapeDtypeStruct((NW, npad), jnp.float32),
        mesh=_sc_mesh(),
        scratch_types=[
            pltpu.VMEM((epw,), jnp.int32),
            pltpu.VMEM((npad,), jnp.float32),
        ],
        compiler_params=pltpu.CompilerParams(needs_layout_passes=False),
    )
    def deg_kernel(ei_hbm, pad_hbm, out_hbm, dst_v, hist_v):
        c = lax.axis_index("c")
        s = lax.axis_index("s")
        w = c * NS + s
        for h in range(nhalf):
            wi = w * nhalf + h
            seg = dst_v.at[pl.ds(h * win, win)]

            @pl.when(wi < nreal_win)
            def _():
                pltpu.sync_copy(ei_hbm.at[1, pl.ds(wi * win, win)], seg)

            @pl.when(wi >= nreal_win)
            def _():
                pltpu.sync_copy(
                    pad_hbm.at[pl.ds((wi - nreal_win) * win, win)], seg)

        def zero_body(i, carry):
            hist_v[pl.ds(i * LANES, LANES)] = jnp.zeros((LANES,), jnp.float32)
            return carry

        lax.fori_loop(0, npad // LANES, zero_body, 0)

        ones = jnp.ones((LANES,), jnp.float32)

        def hist_body(i, carry):
            idx = dst_v[pl.ds(i * LANES, LANES)]
            plsc.addupdate_scatter(hist_v, [idx], ones)
            return carry

        lax.fori_loop(0, epw // LANES, hist_body, 0)
        pltpu.sync_copy(hist_v, out_hbm.at[w])

    return deg_kernel


def _make_agg_kernel(epw, npad, dim, chunk, nbuf, nhalf, e):
    stripe = npad // NS
    nch2 = epw // chunk // nhalf
    win = epw // nhalf
    nreal_win = e // win

    @functools.partial(
        pl.kernel,
        out_type=jax.ShapeDtypeStruct((NC, npad, dim), jnp.float32),
        mesh=_sc_mesh(),
        scratch_types=[
            pltpu.VMEM((nch2 * chunk,), jnp.int32),
            pltpu.VMEM((nch2, chunk), jnp.int32),
            [pltpu.VMEM((chunk, dim), jnp.float32) for _ in range(nbuf)],
            pltpu.VMEM_SHARED((npad, dim), jnp.float32),
            [pltpu.SemaphoreType.DMA for _ in range(nbuf)],
        ],
        compiler_params=pltpu.CompilerParams(needs_layout_passes=False),
    )
    def agg_kernel(g_hbm, src_hbm, dst3_hbm, padf_hbm, pad3_hbm, out_hbm,
                   sidx, didx, bufs, acc, sems):
        c = lax.axis_index("c")
        s = lax.axis_index("s")
        w = c * NS + s
        rows0 = bufs[0]

        def zrow(t, carry):
            rows0[t // (dim // LANES), pl.ds((t % (dim // LANES)) * LANES, LANES)] = (
                jnp.zeros((LANES,), jnp.float32)
            )
            return carry

        lax.fori_loop(0, chunk * (dim // LANES), zrow, 0)

        def zstripe(j, carry):
            pltpu.sync_copy(rows0, acc.at[pl.ds(s * stripe + j * chunk, chunk)])
            return carry

        lax.fori_loop(0, stripe // chunk, zstripe, 0)
        plsc.subcore_barrier()

        for h in range(nhalf):
            wi = w * nhalf + h

            @pl.when(wi < nreal_win)
            def _():
                pltpu.sync_copy(src_hbm.at[pl.ds(wi * win, win)], sidx)
                pltpu.sync_copy(dst3_hbm.at[pl.ds(wi * nch2, nch2)], didx)

            @pl.when(wi >= nreal_win)
            def _():
                pltpu.sync_copy(
                    padf_hbm.at[pl.ds((wi - nreal_win) * win, win)], sidx)
                pltpu.sync_copy(pad3_hbm.at[wi - nreal_win], didx)

            def gidx(i):
                return sidx.at[pl.ds(i * chunk, chunk)]

            for b in range(nbuf):
                pltpu.async_copy(g_hbm.at[gidx(b)], bufs[b], sems[b])

            def chunk_body(j, carry):
                for b in range(nbuf):
                    i = j * nbuf + b
                    pltpu.make_async_copy(
                        g_hbm.at[gidx(i)], bufs[b], sems[b]).wait()
                    pltpu.sync_copy(bufs[b], acc.at[didx.at[i]], add=True)

                    @pl.when(i + nbuf < nch2)
                    def _():
                        pltpu.async_copy(
                            g_hbm.at[gidx(i + nbuf)], bufs[b], sems[b])
                return carry

            lax.fori_loop(0, nch2 // nbuf, chunk_body, 0)
        plsc.subcore_barrier()
        pltpu.sync_copy(
            acc.at[pl.ds(s * stripe, stripe)],
            out_hbm.at[c, pl.ds(s * stripe, stripe)],
        )

    return agg_kernel


def _dinv_from_hist(hist_blk):
    deg = jnp.sum(hist_blk, axis=0) + 1.0
    return lax.rsqrt(deg)


def _make_tc_first(n):
    def _tc_first(hist_ref, x_ref, w_ref, g_ref):
        dinv = _dinv_from_hist(hist_ref[...])
        h = jnp.dot(x_ref[...], w_ref[...], preferred_element_type=jnp.float32)
        rid = (pl.program_id(0) * ROWBLK
               + jax.lax.broadcasted_iota(jnp.int32, (ROWBLK, 1), 0))
        g_ref[...] = jnp.where(rid < n, h * dinv[:, None], 0.0)

    return _tc_first


def _tc_mid(hist_ref, s_ref, g_ref, b_ref, w_ref, g2_ref):
    dinv = _dinv_from_hist(hist_ref[...])
    agg = s_ref[0] + s_ref[1] + g_ref[...]
    act = jnp.maximum(agg * dinv[:, None] + b_ref[...], 0.0)
    h = jnp.dot(act, w_ref[...], preferred_element_type=jnp.float32)
    g2_ref[...] = h * dinv[:, None]


def _tc_last(hist_ref, s_ref, g_ref, b_ref, out_ref):
    dinv = _dinv_from_hist(hist_ref[...])
    agg = s_ref[0] + s_ref[1] + g_ref[...]
    out_ref[...] = agg * dinv[:, None] + b_ref[...]


def kernel(x, edge_index, W1, b1, W2, b2):
    n, dim = x.shape
    e = edge_index.shape[1]
    npad = ((n + ROWBLK) // ROWBLK) * ROWBLK
    chunk, nbuf, nhalf = 64, 4, 4
    chw = nhalf * nbuf * chunk
    epw = ((e + NW * chw - 1) // (NW * chw)) * chw
    epad = NW * epw
    grid = npad // ROWBLK

    win = epw // nhalf
    assert e % win == 0 and e % chunk == 0 and (e // NW) % 8 == 0
    pad_tail = n + jnp.arange(epad - e, dtype=jnp.int32) % (npad - n)
    pad3 = pad_tail.reshape((epad - e) // win, win // chunk, chunk)
    src_flat = edge_index[0]
    dst3 = edge_index[1].reshape(e // chunk, chunk)
    b1r = b1.reshape(1, dim)
    b2r = b2.reshape(1, dim)

    hist = _make_deg_kernel(epw, npad, e, nhalf)(edge_index, pad_tail)
    agg = _make_agg_kernel(epw, npad, dim, chunk, nbuf, nhalf, e)

    hist_spec = pl.BlockSpec((NW, ROWBLK), lambda i: (0, i))
    row_spec = pl.BlockSpec((ROWBLK, dim), lambda i: (i, 0))
    s_spec = pl.BlockSpec((NC, ROWBLK, dim), lambda i: (0, i, 0))
    full_spec = pl.BlockSpec((dim, dim), lambda i: (0, 0))
    bias_spec = pl.BlockSpec((1, dim), lambda i: (0, 0))
    out_type = jax.ShapeDtypeStruct((npad, dim), jnp.float32)

    g1 = pl.pallas_call(
        _make_tc_first(n),
        grid=(grid,),
        in_specs=[hist_spec, row_spec, full_spec],
        out_specs=row_spec,
        out_shape=out_type,
    )(hist, x, W1)

    s1 = agg(g1, src_flat, dst3, pad_tail, pad3)

    g2 = pl.pallas_call(
        _tc_mid,
        grid=(grid,),
        in_specs=[hist_spec, s_spec, row_spec, bias_spec, full_spec],
        out_specs=row_spec,
        out_shape=out_type,
    )(hist, s1, g1, b1r, W2)

    s2 = agg(g2, src_flat, dst3, pad_tail, pad3)

    out = pl.pallas_call(
        _tc_last,
        grid=(grid,),
        in_specs=[hist_spec, s_spec, row_spec, bias_spec],
        out_specs=row_spec,
        out_shape=jax.ShapeDtypeStruct((n, dim), jnp.float32),
    )(hist, s2, g2, b2r)

    return out


# speedup vs baseline: 1.0185x; 1.0105x over previous
import functools

import jax
import jax.numpy as jnp
from jax import lax
from jax.experimental import pallas as pl
from jax.experimental.pallas import tpu as pltpu
from jax.experimental.pallas import tpu_sc as plsc

NC, NS, LANES = 2, 16, 16
NW = NC * NS
CHUNK = 128
ROWBLK = 1024


def _sc_mesh():
    return plsc.VectorSubcoreMesh(core_axis_name="c", subcore_axis_name="s")


def _make_deg_kernel(epw, npad, e, nhalf):
    win = epw // nhalf
    nreal_win = e // win

    @functools.partial(
        pl.kernel,
        out_type=jax.ShapeDtypeStruct((NW, npad), jnp.float32),
        mesh=_sc_mesh(),
        scratch_types=[
            pltpu.VMEM((epw,), jnp.int32),
            pltpu.VMEM((npad,), jnp.float32),
        ],
        compiler_params=pltpu.CompilerParams(needs_layout_passes=False),
    )
    def deg_kernel(ei_hbm, pad_hbm, out_hbm, dst_v, hist_v):
        c = lax.axis_index("c")
        s = lax.axis_index("s")
        w = c * NS + s
        for h in range(nhalf):
            wi = w * nhalf + h
            seg = dst_v.at[pl.ds(h * win, win)]

            @pl.when(wi < nreal_win)
            def _():
                pltpu.sync_copy(ei_hbm.at[1, pl.ds(wi * win, win)], seg)

            @pl.when(wi >= nreal_win)
            def _():
                pltpu.sync_copy(
                    pad_hbm.at[pl.ds((wi - nreal_win) * win, win)], seg)

        def zero_body(i, carry):
            hist_v[pl.ds(i * LANES, LANES)] = jnp.zeros((LANES,), jnp.float32)
            return carry

        lax.fori_loop(0, npad // LANES, zero_body, 0)

        ones = jnp.ones((LANES,), jnp.float32)

        def hist_body(i, carry):
            idx = dst_v[pl.ds(i * LANES, LANES)]
            plsc.addupdate_scatter(hist_v, [idx], ones)
            return carry

        lax.fori_loop(0, epw // LANES, hist_body, 0)
        pltpu.sync_copy(hist_v, out_hbm.at[w])

    return deg_kernel


def _make_agg_kernel(epw, npad, dim, chunk, nbuf, nhalf, e):
    stripe = npad // NS
    nch2 = epw // chunk // nhalf
    win = epw // nhalf
    nreal_win = e // win

    @functools.partial(
        pl.kernel,
        out_type=jax.ShapeDtypeStruct((NC, npad, dim), jnp.float32),
        mesh=_sc_mesh(),
        scratch_types=[
            pltpu.VMEM((nch2, chunk), jnp.int32),
            pltpu.VMEM((nch2, chunk), jnp.int32),
            [pltpu.VMEM((chunk, dim), jnp.float32) for _ in range(nbuf)],
            pltpu.VMEM_SHARED((npad, dim), jnp.float32),
            [pltpu.SemaphoreType.DMA for _ in range(nbuf)],
        ],
        compiler_params=pltpu.CompilerParams(needs_layout_passes=False),
    )
    def agg_kernel(g_hbm, ei_hbm, pad_hbm, out_hbm,
                   sidx, didx, bufs, acc, sems):
        c = lax.axis_index("c")
        s = lax.axis_index("s")
        w = c * NS + s
        rows0 = bufs[0]

        def zrow(t, carry):
            rows0[t // (dim // LANES), pl.ds((t % (dim // LANES)) * LANES, LANES)] = (
                jnp.zeros((LANES,), jnp.float32)
            )
            return carry

        lax.fori_loop(0, chunk * (dim // LANES), zrow, 0)

        def zstripe(j, carry):
            pltpu.sync_copy(rows0, acc.at[pl.ds(s * stripe + j * chunk, chunk)])
            return carry

        lax.fori_loop(0, stripe // chunk, zstripe, 0)
        plsc.subcore_barrier()

        for h in range(nhalf):
            wi = w * nhalf + h

            @pl.when(wi < nreal_win)
            def _():
                row0 = wi * nch2
                pltpu.sync_copy(ei_hbm.at[0, pl.ds(row0, nch2)], sidx)
                pltpu.sync_copy(ei_hbm.at[1, pl.ds(row0, nch2)], didx)

            @pl.when(wi >= nreal_win)
            def _():
                pltpu.sync_copy(pad_hbm.at[wi - nreal_win], sidx)
                pltpu.sync_copy(pad_hbm.at[wi - nreal_win], didx)

            for b in range(nbuf):
                pltpu.async_copy(g_hbm.at[sidx.at[b]], bufs[b], sems[b])

            def chunk_body(j, carry):
                for b in range(nbuf):
                    i = j * nbuf + b
                    pltpu.make_async_copy(
                        g_hbm.at[sidx.at[i]], bufs[b], sems[b]).wait()
                    pltpu.sync_copy(bufs[b], acc.at[didx.at[i]], add=True)

                    @pl.when(i + nbuf < nch2)
                    def _():
                        pltpu.async_copy(
                            g_hbm.at[sidx.at[i + nbuf]], bufs[b], sems[b])
                return carry

            lax.fori_loop(0, nch2 // nbuf, chunk_body, 0)
        plsc.subcore_barrier()
        pltpu.sync_copy(
            acc.at[pl.ds(s * stripe, stripe)],
            out_hbm.at[c, pl.ds(s * stripe, stripe)],
        )

    return agg_kernel


def _dinv_from_hist(hist_blk):
    deg = jnp.sum(hist_blk, axis=0) + 1.0
    return lax.rsqrt(deg)


def _make_tc_first(n):
    def _tc_first(hist_ref, x_ref, w_ref, g_ref):
        dinv = _dinv_from_hist(hist_ref[...])
        h = jnp.dot(x_ref[...], w_ref[...], preferred_element_type=jnp.float32)
        rid = (pl.program_id(0) * ROWBLK
               + jax.lax.broadcasted_iota(jnp.int32, (ROWBLK, 1), 0))
        g_ref[...] = jnp.where(rid < n, h * dinv[:, None], 0.0)

    return _tc_first


def _tc_mid(hist_ref, s_ref, g_ref, b_ref, w_ref, g2_ref):
    dinv = _dinv_from_hist(hist_ref[...])
    agg = s_ref[0] + s_ref[1] + g_ref[...]
    act = jnp.maximum(agg * dinv[:, None] + b_ref[...], 0.0)
    h = jnp.dot(act, w_ref[...], preferred_element_type=jnp.float32)
    g2_ref[...] = h * dinv[:, None]


def _tc_last(hist_ref, s_ref, g_ref, b_ref, out_ref):
    dinv = _dinv_from_hist(hist_ref[...])
    agg = s_ref[0] + s_ref[1] + g_ref[...]
    out_ref[...] = agg * dinv[:, None] + b_ref[...]


def kernel(x, edge_index, W1, b1, W2, b2):
    n, dim = x.shape
    e = edge_index.shape[1]
    npad = ((n + ROWBLK) // ROWBLK) * ROWBLK
    chunk, nbuf, nhalf = 64, 4, 4
    chw = nhalf * nbuf * chunk
    epw = ((e + NW * chw - 1) // (NW * chw)) * chw
    epad = NW * epw
    grid = npad // ROWBLK

    win = epw // nhalf
    assert e % win == 0 and e % chunk == 0 and (e // NW) % 8 == 0
    pad_tail = n + jnp.arange(epad - e, dtype=jnp.int32) % (npad - n)
    pad3 = pad_tail.reshape((epad - e) // win, win // chunk, chunk)
    ei3 = edge_index.reshape(2, e // chunk, chunk)
    b1r = b1.reshape(1, dim)
    b2r = b2.reshape(1, dim)

    hist = _make_deg_kernel(epw, npad, e, nhalf)(edge_index, pad_tail)
    agg = _make_agg_kernel(epw, npad, dim, chunk, nbuf, nhalf, e)

    hist_spec = pl.BlockSpec((NW, ROWBLK), lambda i: (0, i))
    row_spec = pl.BlockSpec((ROWBLK, dim), lambda i: (i, 0))
    s_spec = pl.BlockSpec((NC, ROWBLK, dim), lambda i: (0, i, 0))
    full_spec = pl.BlockSpec((dim, dim), lambda i: (0, 0))
    bias_spec = pl.BlockSpec((1, dim), lambda i: (0, 0))
    out_type = jax.ShapeDtypeStruct((npad, dim), jnp.float32)

    g1 = pl.pallas_call(
        _make_tc_first(n),
        grid=(grid,),
        in_specs=[hist_spec, row_spec, full_spec],
        out_specs=row_spec,
        out_shape=out_type,
    )(hist, x, W1)

    s1 = agg(g1, ei3, pad3)

    g2 = pl.pallas_call(
        _tc_mid,
        grid=(grid,),
        in_specs=[hist_spec, s_spec, row_spec, bias_spec, full_spec],
        out_specs=row_spec,
        out_shape=out_type,
    )(hist, s1, g1, b1r, W2)

    s2 = agg(g2, ei3, pad3)

    out = pl.pallas_call(
        _tc_last,
        grid=(grid,),
        in_specs=[hist_spec, s_spec, row_spec, bias_spec],
        out_specs=row_spec,
        out_shape=jax.ShapeDtypeStruct((n, dim), jnp.float32),
    )(hist, s2, g2, b2r)

    return out


# speedup vs baseline: 1.0185x; 1.0000x over previous
import functools

import jax
import jax.numpy as jnp
from jax import lax
from jax.experimental import pallas as pl
from jax.experimental.pallas import tpu as pltpu
from jax.experimental.pallas import tpu_sc as plsc

NC, NS, LANES = 2, 16, 16
NW = NC * NS
ROWBLK = 1024


def _sc_mesh():
    return plsc.VectorSubcoreMesh(core_axis_name="c", subcore_axis_name="s")


def _make_deg_kernel(epw, npad, e, nhalf):
    win = epw // nhalf
    nreal_win = e // win

    @functools.partial(
        pl.kernel,
        out_type=jax.ShapeDtypeStruct((NW, npad), jnp.float32),
        mesh=_sc_mesh(),
        scratch_types=[
            pltpu.VMEM((epw,), jnp.int32),
            pltpu.VMEM((npad,), jnp.float32),
        ],
        compiler_params=pltpu.CompilerParams(needs_layout_passes=False),
    )
    def deg_kernel(ei_hbm, pad_hbm, out_hbm, dst_v, hist_v):
        c = lax.axis_index("c")
        s = lax.axis_index("s")
        w = c * NS + s
        for h in range(nhalf):
            wi = w * nhalf + h
            seg = dst_v.at[pl.ds(h * win, win)]

            @pl.when(wi < nreal_win)
            def _():
                pltpu.sync_copy(ei_hbm.at[1, pl.ds(wi * win, win)], seg)

            @pl.when(wi >= nreal_win)
            def _():
                pltpu.sync_copy(
                    pad_hbm.at[pl.ds((wi - nreal_win) * win, win)], seg)

        def zero_body(i, carry):
            hist_v[pl.ds(i * LANES, LANES)] = jnp.zeros((LANES,), jnp.float32)
            return carry

        lax.fori_loop(0, npad // LANES, zero_body, 0)

        ones = jnp.ones((LANES,), jnp.float32)

        def hist_body(i, carry):
            idx = dst_v[pl.ds(i * LANES, LANES)]
            plsc.addupdate_scatter(hist_v, [idx], ones)
            return carry

        lax.fori_loop(0, epw // LANES, hist_body, 0)
        pltpu.sync_copy(hist_v, out_hbm.at[w])

    return deg_kernel


def _make_agg_kernel(epw, npad, dim, chunk, nbuf, nhalf, e):
    stripe = npad // NS
    nch2 = epw // chunk // nhalf
    win = epw // nhalf
    nreal_win = e // win

    @functools.partial(
        pl.kernel,
        out_type=jax.ShapeDtypeStruct((NC, npad, dim), jnp.float32),
        mesh=_sc_mesh(),
        scratch_types=[
            pltpu.VMEM((nch2, chunk), jnp.int32),
            pltpu.VMEM((nch2, chunk), jnp.int32),
            [pltpu.VMEM((chunk, dim), jnp.float32) for _ in range(nbuf)],
            pltpu.VMEM_SHARED((npad, dim), jnp.float32),
            [pltpu.SemaphoreType.DMA for _ in range(nbuf)],
        ],
        compiler_params=pltpu.CompilerParams(needs_layout_passes=False),
    )
    def agg_kernel(g_hbm, ei_hbm, pad_hbm, out_hbm,
                   sidx, didx, bufs, acc, sems):
        c = lax.axis_index("c")
        s = lax.axis_index("s")
        w = c * NS + s
        rows0 = bufs[0]

        def zrow(t, carry):
            rows0[t // (dim // LANES), pl.ds((t % (dim // LANES)) * LANES, LANES)] = (
                jnp.zeros((LANES,), jnp.float32)
            )
            return carry

        lax.fori_loop(0, chunk * (dim // LANES), zrow, 0)

        def zstripe(j, carry):
            pltpu.sync_copy(rows0, acc.at[pl.ds(s * stripe + j * chunk, chunk)])
            return carry

        lax.fori_loop(0, stripe // chunk, zstripe, 0)
        plsc.subcore_barrier()

        for h in range(nhalf):
            wi = w * nhalf + h

            @pl.when(wi < nreal_win)
            def _():
                row0 = wi * nch2
                pltpu.sync_copy(ei_hbm.at[0, pl.ds(row0, nch2)], sidx)
                pltpu.sync_copy(ei_hbm.at[1, pl.ds(row0, nch2)], didx)

            @pl.when(wi >= nreal_win)
            def _():
                pltpu.sync_copy(pad_hbm.at[wi - nreal_win], sidx)
                pltpu.sync_copy(pad_hbm.at[wi - nreal_win], didx)

            for b in range(nbuf):
                pltpu.async_copy(g_hbm.at[sidx.at[b]], bufs[b], sems[b])

            def chunk_body(j, carry):
                for b in range(nbuf):
                    i = j * nbuf + b
                    pltpu.make_async_copy(
                        g_hbm.at[sidx.at[i]], bufs[b], sems[b]).wait()
                    pltpu.sync_copy(bufs[b], acc.at[didx.at[i]], add=True)

                    @pl.when(i + nbuf < nch2)
                    def _():
                        pltpu.async_copy(
                            g_hbm.at[sidx.at[i + nbuf]], bufs[b], sems[b])
                return carry

            lax.fori_loop(0, nch2 // nbuf, chunk_body, 0)
        plsc.subcore_barrier()
        pltpu.sync_copy(
            acc.at[pl.ds(s * stripe, stripe)],
            out_hbm.at[c, pl.ds(s * stripe, stripe)],
        )

    return agg_kernel


def _dinv_from_hist(hist_blk):
    deg = jnp.sum(hist_blk, axis=0) + 1.0
    return lax.rsqrt(deg)


def _make_tc_first(n):
    def _tc_first(hist_ref, x_ref, w_ref, g_ref):
        dinv = _dinv_from_hist(hist_ref[...])
        h = jnp.dot(x_ref[...], w_ref[...], preferred_element_type=jnp.float32)
        rid = (pl.program_id(0) * ROWBLK
               + jax.lax.broadcasted_iota(jnp.int32, (ROWBLK, 1), 0))
        g_ref[...] = jnp.where(rid < n, h * dinv[:, None], 0.0)

    return _tc_first


def _tc_mid(hist_ref, s_ref, g_ref, b_ref, w_ref, g2_ref):
    dinv = _dinv_from_hist(hist_ref[...])
    agg = s_ref[0] + s_ref[1] + g_ref[...]
    act = jnp.maximum(agg * dinv[:, None] + b_ref[...], 0.0)
    h = jnp.dot(act, w_ref[...], preferred_element_type=jnp.float32)
    g2_ref[...] = h * dinv[:, None]


def _tc_last(hist_ref, s_ref, g_ref, b_ref, out_ref):
    dinv = _dinv_from_hist(hist_ref[...])
    agg = s_ref[0] + s_ref[1] + g_ref[...]
    out_ref[...] = agg * dinv[:, None] + b_ref[...]


def kernel(x, edge_index, W1, b1, W2, b2):
    n, dim = x.shape
    e = edge_index.shape[1]
    npad = ((n + ROWBLK) // ROWBLK) * ROWBLK
    chunk, nbuf, nhalf = 64, 4, 4
    chw = nhalf * nbuf * chunk
    epw = ((e + NW * chw - 1) // (NW * chw)) * chw
    epad = NW * epw
    grid = npad // ROWBLK

    win = epw // nhalf
    assert e % win == 0 and e % chunk == 0 and (e // NW) % 8 == 0
    pad_tail = n + jnp.arange(epad - e, dtype=jnp.int32) % (npad - n)
    pad3 = pad_tail.reshape((epad - e) // win, win // chunk, chunk)
    ei3 = edge_index.reshape(2, e // chunk, chunk)
    b1r = b1.reshape(1, dim)
    b2r = b2.reshape(1, dim)

    hist = _make_deg_kernel(epw, npad, e, nhalf)(edge_index, pad_tail)
    agg = _make_agg_kernel(epw, npad, dim, chunk, nbuf, nhalf, e)

    hist_spec = pl.BlockSpec((NW, ROWBLK), lambda i: (0, i))
    row_spec = pl.BlockSpec((ROWBLK, dim), lambda i: (i, 0))
    s_spec = pl.BlockSpec((NC, ROWBLK, dim), lambda i: (0, i, 0))
    full_spec = pl.BlockSpec((dim, dim), lambda i: (0, 0))
    bias_spec = pl.BlockSpec((1, dim), lambda i: (0, 0))
    out_type = jax.ShapeDtypeStruct((npad, dim), jnp.float32)

    g1 = pl.pallas_call(
        _make_tc_first(n),
        grid=(grid,),
        in_specs=[hist_spec, row_spec, full_spec],
        out_specs=row_spec,
        out_shape=out_type,
    )(hist, x, W1)

    s1 = agg(g1, ei3, pad3)

    g2 = pl.pallas_call(
        _tc_mid,
        grid=(grid,),
        in_specs=[hist_spec, s_spec, row_spec, bias_spec, full_spec],
        out_specs=row_spec,
        out_shape=out_type,
    )(hist, s1, g1, b1r, W2)

    s2 = agg(g2, ei3, pad3)

    out = pl.pallas_call(
        _tc_last,
        grid=(grid,),
        in_specs=[hist_spec, s_spec, row_spec, bias_spec],
        out_specs=row_spec,
        out_shape=jax.ShapeDtypeStruct((n, dim), jnp.float32),
    )(hist, s2, g2, b2r)

    return out
